# baseline (speedup 1.0000x reference)
import functools

import jax
import jax.numpy as jnp
from jax import lax
from jax.experimental import pallas as pl
from jax.experimental.pallas import tpu as pltpu
from jax.experimental.pallas import tpu_sc as plsc

D_MODEL = 64
SAMPLE = 200
GATHER_CHUNK = 100
N_CHUNKS = SAMPLE // GATHER_CHUNK
N_GROUPS = 13
TAIL_START = SAMPLE - 16


def _make_kernel(batch, n_per_worker):
    mesh = plsc.VectorSubcoreMesh(core_axis_name="c", subcore_axis_name="s")
    num_cores = 2

    @functools.partial(
        pl.kernel,
        out_type=jax.ShapeDtypeStruct((batch, SAMPLE), jnp.float32),
        mesh=mesh,
        compiler_params=pltpu.CompilerParams(
            needs_layout_passes=False, use_tc_tiling_on_sc=False),
        scratch_types=[
            pltpu.VMEM((N_CHUNKS, GATHER_CHUNK), jnp.int32),
            pltpu.VMEM((SAMPLE, D_MODEL), jnp.float32),
            pltpu.VMEM((D_MODEL,), jnp.float32),
            pltpu.VMEM((16, 16), jnp.float32),
            pltpu.VMEM((SAMPLE,), jnp.float32),
            pltpu.SemaphoreType.DMA,
        ],
    )
    def emb_dot(h_hbm, idx_hbm, w_hbm, out_hbm, idx_v, rows_v, h_v, cum_v,
                out_v, sem):
        wid = lax.axis_index("s") * num_cores + lax.axis_index("c")
        base_b = wid * n_per_worker

        lane = lax.iota(jnp.int32, 16)

        @pl.loop(0, n_per_worker)
        def per_b(k):
            b = base_b + k
            pltpu.sync_copy(idx_hbm.at[b], idx_v)
            copies = [
                pltpu.async_copy(
                    w_hbm.at[idx_v.at[j]],
                    rows_v.at[pl.ds(j * GATHER_CHUNK, GATHER_CHUNK)],
                    sem,
                )
                for j in range(N_CHUNKS)
            ]
            pltpu.sync_copy(h_hbm.at[b], h_v)
            for c in copies:
                c.wait()

            h_chunk = [h_v[pl.ds(16 * c, 16)] for c in range(4)]

            @pl.loop(0, N_GROUPS)
            def per_group(g):
                s0 = jnp.where(g < N_GROUPS - 1, g * 16, TAIL_START)
                for j in range(16):
                    s = s0 + j
                    t = rows_v[s, pl.ds(0, 16)] * h_chunk[0]
                    for c in range(1, 4):
                        t = t + rows_v[s, pl.ds(16 * c, 16)] * h_chunk[c]
                    cum_v[j] = t
                out_g = plsc.load_gather(
                    cum_v, [lane, jnp.zeros((16,), jnp.int32)])
                for i in range(1, 16):
                    out_g = out_g + plsc.load_gather(
                        cum_v, [lane, jnp.full((16,), i, jnp.int32)])
                out_v[pl.ds(s0, 16)] = out_g

            pltpu.sync_copy(out_v, out_hbm.at[b])

    return emb_dot


@jax.jit
def kernel(h, indicies, W):
    batch = h.shape[0]
    n_workers = 32
    h2 = jnp.reshape(h, (batch, D_MODEL))
    idx = jnp.reshape(indicies.astype(jnp.int32), (batch, N_CHUNKS, GATHER_CHUNK))
    out = _make_kernel(batch, batch // n_workers)(h2, idx, W)
    return jnp.reshape(out, (batch, 1, SAMPLE))


# speedup vs baseline: 1.4468x; 1.4468x over previous
import functools

import jax
import jax.numpy as jnp
from jax import lax
from jax.experimental import pallas as pl
from jax.experimental.pallas import tpu as pltpu
from jax.experimental.pallas import tpu_sc as plsc

D_MODEL = 64
SAMPLE = 200
GATHER_CHUNK = 100
N_CHUNKS = SAMPLE // GATHER_CHUNK
N_GROUPS = 13
TAIL_START = SAMPLE - 16


def _make_kernel(batch, n_per_worker):
    mesh = plsc.VectorSubcoreMesh(core_axis_name="c", subcore_axis_name="s")
    num_cores = 2

    @functools.partial(
        pl.kernel,
        out_type=jax.ShapeDtypeStruct((batch, SAMPLE), jnp.float32),
        mesh=mesh,
        compiler_params=pltpu.CompilerParams(
            needs_layout_passes=False, use_tc_tiling_on_sc=False),
        scratch_types=[
            pltpu.VMEM((2, N_CHUNKS, GATHER_CHUNK), jnp.int32),
            pltpu.VMEM((2, SAMPLE, D_MODEL), jnp.float32),
            pltpu.VMEM((2, D_MODEL), jnp.float32),
            pltpu.VMEM((16, 16), jnp.float32),
            pltpu.VMEM((2, SAMPLE), jnp.float32),
            pltpu.SemaphoreType.DMA,
            pltpu.SemaphoreType.DMA,
            pltpu.SemaphoreType.DMA,
            pltpu.SemaphoreType.DMA,
            pltpu.SemaphoreType.DMA,
            pltpu.SemaphoreType.DMA,
            pltpu.SemaphoreType.DMA,
            pltpu.SemaphoreType.DMA,
        ],
    )
    def emb_dot(h_hbm, idx_hbm, w_hbm, out_hbm, idx_v, rows_v, h_v, cum_v,
                out_v, idx_sem0, idx_sem1, rows_sem0, rows_sem1, h_sem0,
                h_sem1, out_sem0, out_sem1):
        wid = lax.axis_index("s") * num_cores + lax.axis_index("c")
        base_b = wid * n_per_worker
        idx_sem = (idx_sem0, idx_sem1)
        rows_sem = (rows_sem0, rows_sem1)
        h_sem = (h_sem0, h_sem1)
        out_sem = (out_sem0, out_sem1)

        lane = lax.iota(jnp.int32, 16)

        def issue_idx(k, slot):
            pltpu.async_copy(idx_hbm.at[base_b + k], idx_v.at[slot],
                             idx_sem[slot])

        def issue_rows(k, slot):
            for j in range(N_CHUNKS):
                pltpu.async_copy(
                    w_hbm.at[idx_v.at[slot, j]],
                    rows_v.at[slot, pl.ds(j * GATHER_CHUNK, GATHER_CHUNK)],
                    rows_sem[slot])

        def issue_h(k, slot):
            pltpu.async_copy(h_hbm.at[base_b + k], h_v.at[slot], h_sem[slot])

        def drain(dummy_hbm_src, dst_ref, sem):
            pltpu.make_async_copy(dummy_hbm_src, dst_ref, sem).wait()

        def compute(slot):
            rows = rows_v.at[slot]
            h_chunk = [h_v[slot, pl.ds(16 * c, 16)] for c in range(4)]

            @pl.loop(0, N_GROUPS)
            def per_group(g):
                s0 = jnp.where(g < N_GROUPS - 1, g * 16, TAIL_START)
                for j in range(16):
                    s = s0 + j
                    t = rows[s, pl.ds(0, 16)] * h_chunk[0]
                    for c in range(1, 4):
                        t = t + rows[s, pl.ds(16 * c, 16)] * h_chunk[c]
                    cum_v[j] = t
                out_g = plsc.load_gather(
                    cum_v, [lane, jnp.zeros((16,), jnp.int32)])
                for i in range(1, 16):
                    out_g = out_g + plsc.load_gather(
                        cum_v, [lane, jnp.full((16,), i, jnp.int32)])
                out_v[slot, pl.ds(s0, 16)] = out_g

        def step(k, slot):
            @pl.when(k >= 2)
            def _():
                drain(out_hbm.at[base_b], out_v.at[slot], out_sem[slot])

            drain(w_hbm.at[pl.ds(0, SAMPLE)], rows_v.at[slot],
                  rows_sem[slot])
            drain(h_hbm.at[base_b], h_v.at[slot], h_sem[slot])

            @pl.when(k < n_per_worker - 2)
            def _():
                issue_idx(k + 2, slot)

            @pl.when(k < n_per_worker - 1)
            def _():
                drain(idx_hbm.at[base_b], idx_v.at[1 - slot],
                      idx_sem[1 - slot])
                issue_rows(k + 1, 1 - slot)
                issue_h(k + 1, 1 - slot)

            compute(slot)

            pltpu.async_copy(out_v.at[slot], out_hbm.at[base_b + k],
                             out_sem[slot])

        issue_idx(0, 0)
        issue_idx(1, 1)
        issue_h(0, 0)
        drain(idx_hbm.at[base_b], idx_v.at[0], idx_sem[0])
        issue_rows(0, 0)

        @pl.loop(0, n_per_worker, step=2)
        def per_pair(k):
            step(k, 0)
            step(k + 1, 1)

        drain(out_hbm.at[base_b], out_v.at[0], out_sem[0])
        drain(out_hbm.at[base_b], out_v.at[1], out_sem[1])

    return emb_dot


@jax.jit
def kernel(h, indicies, W):
    batch = h.shape[0]
    n_workers = 32
    h2 = jnp.reshape(h, (batch, D_MODEL))
    idx = jnp.reshape(indicies.astype(jnp.int32), (batch, N_CHUNKS, GATHER_CHUNK))
    out = _make_kernel(batch, batch // n_workers)(h2, idx, W)
    return jnp.reshape(out, (batch, 1, SAMPLE))


# speedup vs baseline: 2.0700x; 1.4307x over previous
import functools

import jax
import jax.numpy as jnp
from jax import lax
from jax.experimental import pallas as pl
from jax.experimental.pallas import tpu as pltpu
from jax.experimental.pallas import tpu_sc as plsc

D_MODEL = 64
SAMPLE = 200
GATHER_CHUNK = 100
N_CHUNKS = SAMPLE // GATHER_CHUNK
N_GROUPS = 13
TAIL_START = SAMPLE - 16


def _make_kernel(batch, n_per_worker):
    mesh = plsc.VectorSubcoreMesh(core_axis_name="c", subcore_axis_name="s")
    num_cores = 2

    @functools.partial(
        pl.kernel,
        out_type=jax.ShapeDtypeStruct((batch, SAMPLE), jnp.float32),
        mesh=mesh,
        compiler_params=pltpu.CompilerParams(
            needs_layout_passes=False, use_tc_tiling_on_sc=False),
        scratch_types=[
            pltpu.VMEM((2, N_CHUNKS, GATHER_CHUNK), jnp.int32),
            pltpu.VMEM((2, SAMPLE, D_MODEL), jnp.float32),
            pltpu.VMEM((2, D_MODEL), jnp.float32),
            pltpu.VMEM((16, 16), jnp.float32),
            pltpu.VMEM((2, SAMPLE), jnp.float32),
            pltpu.SemaphoreType.DMA,
            pltpu.SemaphoreType.DMA,
            pltpu.SemaphoreType.DMA,
            pltpu.SemaphoreType.DMA,
            pltpu.SemaphoreType.DMA,
            pltpu.SemaphoreType.DMA,
            pltpu.SemaphoreType.DMA,
            pltpu.SemaphoreType.DMA,
        ],
    )
    def emb_dot(h_hbm, idx_hbm, w_hbm, out_hbm, idx_v, rows_v, h_v, cum_v,
                out_v, idx_sem0, idx_sem1, rows_sem0, rows_sem1, h_sem0,
                h_sem1, out_sem0, out_sem1):
        wid = lax.axis_index("s") * num_cores + lax.axis_index("c")
        base_b = wid * n_per_worker
        idx_sem = (idx_sem0, idx_sem1)
        rows_sem = (rows_sem0, rows_sem1)
        h_sem = (h_sem0, h_sem1)
        out_sem = (out_sem0, out_sem1)

        lane = lax.iota(jnp.int32, 16)

        def issue_idx(k, slot):
            pltpu.async_copy(idx_hbm.at[base_b + k], idx_v.at[slot],
                             idx_sem[slot])

        def issue_rows(k, slot):
            for j in range(N_CHUNKS):
                pltpu.async_copy(
                    w_hbm.at[idx_v.at[slot, j]],
                    rows_v.at[slot, pl.ds(j * GATHER_CHUNK, GATHER_CHUNK)],
                    rows_sem[slot])

        def issue_h(k, slot):
            pltpu.async_copy(h_hbm.at[base_b + k], h_v.at[slot], h_sem[slot])

        def drain(dummy_hbm_src, dst_ref, sem):
            pltpu.make_async_copy(dummy_hbm_src, dst_ref, sem).wait()

        def compute(slot):
            rows = rows_v.at[slot]
            h_chunk = [h_v[slot, pl.ds(16 * c, 16)] for c in range(4)]

            @pl.loop(0, N_GROUPS)
            def per_group(g):
                s0 = jnp.where(g < N_GROUPS - 1, g * 16, TAIL_START)
                ts = []
                for j in range(16):
                    s = s0 + j
                    l = [rows[s, pl.ds(16 * c, 16)] for c in range(4)]
                    ts.append((l[0] * h_chunk[0] + l[1] * h_chunk[1])
                              + (l[2] * h_chunk[2] + l[3] * h_chunk[3]))
                for j in range(16):
                    cum_v[j] = ts[j]
                accs = [
                    plsc.load_gather(cum_v, [lane, jnp.full((16,), i, jnp.int32)])
                    for i in range(4)
                ]
                for i in range(4, 16):
                    accs[i % 4] = accs[i % 4] + plsc.load_gather(
                        cum_v, [lane, jnp.full((16,), i, jnp.int32)])
                out_g = (accs[0] + accs[1]) + (accs[2] + accs[3])
                out_v[slot, pl.ds(s0, 16)] = out_g

        def step(k, slot):
            @pl.when(k >= 2)
            def _():
                drain(out_hbm.at[base_b], out_v.at[slot], out_sem[slot])

            drain(w_hbm.at[pl.ds(0, SAMPLE)], rows_v.at[slot],
                  rows_sem[slot])
            drain(h_hbm.at[base_b], h_v.at[slot], h_sem[slot])

            @pl.when(k < n_per_worker - 2)
            def _():
                issue_idx(k + 2, slot)

            @pl.when(k < n_per_worker - 1)
            def _():
                drain(idx_hbm.at[base_b], idx_v.at[1 - slot],
                      idx_sem[1 - slot])
                issue_rows(k + 1, 1 - slot)
                issue_h(k + 1, 1 - slot)

            compute(slot)

            pltpu.async_copy(out_v.at[slot], out_hbm.at[base_b + k],
                             out_sem[slot])

        issue_idx(0, 0)
        issue_idx(1, 1)
        issue_h(0, 0)
        drain(idx_hbm.at[base_b], idx_v.at[0], idx_sem[0])
        issue_rows(0, 0)

        @pl.loop(0, n_per_worker, step=2)
        def per_pair(k):
            step(k, 0)
            step(k + 1, 1)

        drain(out_hbm.at[base_b], out_v.at[0], out_sem[0])
        drain(out_hbm.at[base_b], out_v.at[1], out_sem[1])

    return emb_dot


@jax.jit
def kernel(h, indicies, W):
    batch = h.shape[0]
    n_workers = 32
    h2 = jnp.reshape(h, (batch, D_MODEL))
    idx = jnp.reshape(indicies.astype(jnp.int32), (batch, N_CHUNKS, GATHER_CHUNK))
    out = _make_kernel(batch, batch // n_workers)(h2, idx, W)
    return jnp.reshape(out, (batch, 1, SAMPLE))
